# baseline (speedup 1.0000x reference)
import functools

import jax
import jax.numpy as jnp
from jax import lax
from jax.experimental import pallas as pl
from jax.experimental.pallas import tpu as pltpu
from jax.experimental.pallas import tpu_sc as plsc

NC = 2
NS = 16
L = 16
CH = 128
BLK = 16


def _zero_rows(ref, nrows, width):
  zero = jnp.zeros((L,), jnp.float32)

  def body(i, carry):
    for j in range(width // L):
      ref[i, pl.ds(j * L, L)] = zero
    return carry

  lax.fori_loop(0, nrows, body, 0, unroll=4)


def _sc_mesh():
  return plsc.VectorSubcoreMesh(core_axis_name="c", subcore_axis_name="s")


def _make_deg_kernel(npad, cpt):
  @functools.partial(
      pl.kernel,
      out_type=jax.ShapeDtypeStruct((NC * NS, npad), jnp.float32),
      mesh=_sc_mesh(),
      compiler_params=pltpu.CompilerParams(needs_layout_passes=False),
      scratch_types=[
          pltpu.VMEM((cpt, CH), jnp.int32),
          pltpu.VMEM((npad,), jnp.float32),
      ],
  )
  def deg_kernel(col3, degp, colbuf, hist):
    c = lax.axis_index("c")
    s = lax.axis_index("s")
    wid = c * NS + s

    pltpu.sync_copy(col3.at[wid], colbuf)

    zero = jnp.zeros((L,), jnp.float32)

    def zbody(i, carry):
      hist[pl.ds(i * L, L)] = zero
      return carry

    lax.fori_loop(0, npad // L, zbody, 0, unroll=8)

    one = jnp.full((L,), 1.0, jnp.float32)

    def chunk(j, carry):
      for k in range(CH // L):
        idx = colbuf[j, pl.ds(k * L, L)]
        plsc.addupdate_scatter(hist, [idx], one)
      return carry

    lax.fori_loop(0, cpt, chunk, 0)
    pltpu.sync_copy(hist, degp.at[wid])

  return deg_kernel


def _make_agg_kernel(npad, d, cpt):
  rows_per_tile = npad // NS

  @functools.partial(
      pl.kernel,
      out_type=jax.ShapeDtypeStruct((NC * npad, d), jnp.float32),
      mesh=_sc_mesh(),
      scratch_types=[
          pltpu.VMEM((cpt, CH), jnp.int32),
          pltpu.VMEM((cpt, CH), jnp.int32),
          pltpu.VMEM((CH, d), jnp.float32),
          pltpu.VMEM_SHARED((npad, d), jnp.float32),
          pltpu.SemaphoreType.DMA,
      ],
  )
  def agg_kernel(xs_hbm, row4, col4, out, rowbuf, colbuf, rows_v, acc, sem):
    c = lax.axis_index("c")
    s = lax.axis_index("s")
    wid = c * NS + s
    nblk = cpt // BLK

    for b in range(nblk):
      pltpu.sync_copy(row4.at[wid * nblk + b], rowbuf.at[pl.ds(b * BLK, BLK)])
      pltpu.sync_copy(col4.at[wid * nblk + b], colbuf.at[pl.ds(b * BLK, BLK)])

    _zero_rows(rows_v, CH, d)
    zbase = s * rows_per_tile
    for r in range(rows_per_tile // CH):
      pltpu.sync_copy(rows_v, acc.at[pl.ds(zbase + r * CH, CH)])
    plsc.subcore_barrier()

    def chunk(j, carry):
      pltpu.async_copy(xs_hbm.at[rowbuf.at[j]], rows_v, sem).wait()
      pltpu.sync_copy(rows_v, acc.at[colbuf.at[j]], add=True)
      return carry

    lax.fori_loop(0, cpt, chunk, 0)
    plsc.subcore_barrier()

    pltpu.sync_copy(
        acc.at[pl.ds(s * rows_per_tile, rows_per_tile)],
        out.at[pl.ds(c * npad + s * rows_per_tile, rows_per_tile)])

  return agg_kernel


def _scale_kernel(dp_ref, x_ref, xs_ref, ds_ref):
  deg = jnp.sum(dp_ref[...], axis=0)[:, None]
  dis = jnp.where(deg > 0.0, lax.rsqrt(jnp.maximum(deg, 1e-30)), 0.0)
  xs_ref[...] = dis * x_ref[...]
  ds_ref[...] = jnp.broadcast_to(dis, ds_ref.shape)


def _head_kernel(p0_ref, p1_ref, ds_ref, w_ref, b_ref, o_ref):
  agg = ds_ref[:, :1] * (p0_ref[...] + p1_ref[...])
  out = jnp.dot(agg, w_ref[...], preferred_element_type=jnp.float32)
  o_ref[...] = jnp.maximum(out + b_ref[...], 0.0)


def kernel(x, edge_index, W, b):
  n, d = x.shape
  e = edge_index.shape[1]

  npad = -(-(n + 1) // (NS * L)) * (NS * L)
  cpt = -(-e // (NC * NS * CH))
  cpt = -(-cpt // BLK) * BLK
  epad = NC * NS * cpt * CH

  row = edge_index[0].astype(jnp.int32)
  col = edge_index[1].astype(jnp.int32)
  pad = jnp.full((epad - e,), n, jnp.int32)
  row3 = jnp.concatenate([row, pad]).reshape(NC * NS, cpt, CH)
  col3 = jnp.concatenate([col, pad]).reshape(NC * NS, cpt, CH)
  row4 = row3.reshape(NC * NS * (cpt // BLK), BLK, CH)
  col4 = col3.reshape(NC * NS * (cpt // BLK), BLK, CH)
  x_pad = jnp.zeros((npad, d), x.dtype).at[:n].set(x)

  degp = _make_deg_kernel(npad, cpt)(col3)

  bn = 256
  grid = (npad // bn,)
  xs, ds16 = pl.pallas_call(
      _scale_kernel,
      grid=grid,
      in_specs=[
          pl.BlockSpec((NC * NS, bn), lambda i: (0, i)),
          pl.BlockSpec((bn, d), lambda i: (i, 0)),
      ],
      out_specs=[
          pl.BlockSpec((bn, d), lambda i: (i, 0)),
          pl.BlockSpec((bn, L), lambda i: (i, 0)),
      ],
      out_shape=[
          jax.ShapeDtypeStruct((npad, d), jnp.float32),
          jax.ShapeDtypeStruct((npad, L), jnp.float32),
      ],
  )(degp, x_pad)

  parts = _make_agg_kernel(npad, d, cpt)(xs, row4, col4)

  h_pad = pl.pallas_call(
      _head_kernel,
      grid=grid,
      in_specs=[
          pl.BlockSpec((bn, d), lambda i: (i, 0)),
          pl.BlockSpec((bn, d), lambda i: (i, 0)),
          pl.BlockSpec((bn, L), lambda i: (i, 0)),
          pl.BlockSpec((d, d), lambda i: (0, 0)),
          pl.BlockSpec((1, d), lambda i: (0, 0)),
      ],
      out_specs=pl.BlockSpec((bn, d), lambda i: (i, 0)),
      out_shape=jax.ShapeDtypeStruct((npad, d), jnp.float32),
  )(parts[:npad], parts[npad:], ds16, W, b.reshape(1, d))

  return h_pad[:n]


# speedup vs baseline: 1.4243x; 1.4243x over previous
import functools

import jax
import jax.numpy as jnp
from jax import lax
from jax.experimental import pallas as pl
from jax.experimental.pallas import tpu as pltpu
from jax.experimental.pallas import tpu_sc as plsc

NC = 2
NS = 16
L = 16
CH = 128
BLK = 16


def _zero_rows(ref, nrows, width):
  zero = jnp.zeros((L,), jnp.float32)

  def body(i, carry):
    for j in range(width // L):
      ref[i, pl.ds(j * L, L)] = zero
    return carry

  lax.fori_loop(0, nrows, body, 0, unroll=4)


def _sc_mesh():
  return plsc.VectorSubcoreMesh(core_axis_name="c", subcore_axis_name="s")


def _make_deg_kernel(npad, cpt):
  @functools.partial(
      pl.kernel,
      out_type=jax.ShapeDtypeStruct((NC * NS, npad), jnp.float32),
      mesh=_sc_mesh(),
      compiler_params=pltpu.CompilerParams(needs_layout_passes=False),
      scratch_types=[
          pltpu.VMEM((cpt, CH), jnp.int32),
          pltpu.VMEM((npad,), jnp.float32),
      ],
  )
  def deg_kernel(col3, degp, colbuf, hist):
    c = lax.axis_index("c")
    s = lax.axis_index("s")
    wid = c * NS + s

    pltpu.sync_copy(col3.at[wid], colbuf)

    zero = jnp.zeros((L,), jnp.float32)

    def zbody(i, carry):
      hist[pl.ds(i * L, L)] = zero
      return carry

    lax.fori_loop(0, npad // L, zbody, 0, unroll=8)

    one = jnp.full((L,), 1.0, jnp.float32)

    def chunk(j, carry):
      for k in range(CH // L):
        idx = colbuf[j, pl.ds(k * L, L)]
        plsc.addupdate_scatter(hist, [idx], one)
      return carry

    lax.fori_loop(0, cpt, chunk, 0)
    pltpu.sync_copy(hist, degp.at[wid])

  return deg_kernel


def _make_agg_kernel(npad, d, cpt):
  rows_per_tile = npad // NS

  @functools.partial(
      pl.kernel,
      out_type=jax.ShapeDtypeStruct((NC * npad, d), jnp.float32),
      mesh=_sc_mesh(),
      scratch_types=[
          pltpu.VMEM((cpt, CH), jnp.int32),
          pltpu.VMEM((cpt, CH), jnp.int32),
          pltpu.VMEM((CH, d), jnp.float32),
          pltpu.VMEM_SHARED((npad, d), jnp.float32),
          pltpu.SemaphoreType.DMA,
      ],
  )
  def agg_kernel(xs_hbm, row3, col3, out, rowbuf, colbuf, rows_v, acc, sem):
    c = lax.axis_index("c")
    s = lax.axis_index("s")
    wid = c * NS + s

    pltpu.sync_copy(row3.at[wid], rowbuf)
    pltpu.sync_copy(col3.at[wid], colbuf)

    _zero_rows(rows_v, CH, d)
    zbase = s * rows_per_tile
    for r in range(rows_per_tile // CH):
      pltpu.sync_copy(rows_v, acc.at[pl.ds(zbase + r * CH, CH)])
    plsc.subcore_barrier()

    def chunk(j, carry):
      pltpu.async_copy(xs_hbm.at[rowbuf.at[j]], rows_v, sem).wait()
      pltpu.sync_copy(rows_v, acc.at[colbuf.at[j]], add=True)
      return carry

    lax.fori_loop(0, cpt, chunk, 0)
    plsc.subcore_barrier()

    pltpu.sync_copy(
        acc.at[pl.ds(s * rows_per_tile, rows_per_tile)],
        out.at[pl.ds(c * npad + s * rows_per_tile, rows_per_tile)])

  return agg_kernel


def _scale_kernel(dp_ref, x_ref, xs_ref, ds_ref):
  deg = jnp.sum(dp_ref[...], axis=0)[:, None]
  dis = jnp.where(deg > 0.0, lax.rsqrt(jnp.maximum(deg, 1e-30)), 0.0)
  xs_ref[...] = dis * x_ref[...]
  ds_ref[...] = jnp.broadcast_to(dis, ds_ref.shape)


def _head_kernel(p0_ref, p1_ref, ds_ref, w_ref, b_ref, o_ref):
  agg = ds_ref[:, :1] * (p0_ref[...] + p1_ref[...])
  out = jnp.dot(agg, w_ref[...], preferred_element_type=jnp.float32)
  o_ref[...] = jnp.maximum(out + b_ref[...], 0.0)


def kernel(x, edge_index, W, b):
  n, d = x.shape
  e = edge_index.shape[1]

  npad = -(-(n + 1) // (NS * L)) * (NS * L)
  cpt = -(-e // (NC * NS * CH))
  epad = NC * NS * cpt * CH

  row = edge_index[0].astype(jnp.int32)
  col = edge_index[1].astype(jnp.int32)
  pad = jnp.full((epad - e,), n, jnp.int32)
  row3 = jnp.concatenate([row, pad]).reshape(NC * NS, cpt, CH)
  col3 = jnp.concatenate([col, pad]).reshape(NC * NS, cpt, CH)
  x_pad = jnp.zeros((npad, d), x.dtype).at[:n].set(x)

  degp = _make_deg_kernel(npad, cpt)(col3)

  bn = 256
  grid = (npad // bn,)
  xs, ds16 = pl.pallas_call(
      _scale_kernel,
      grid=grid,
      in_specs=[
          pl.BlockSpec((NC * NS, bn), lambda i: (0, i)),
          pl.BlockSpec((bn, d), lambda i: (i, 0)),
      ],
      out_specs=[
          pl.BlockSpec((bn, d), lambda i: (i, 0)),
          pl.BlockSpec((bn, L), lambda i: (i, 0)),
      ],
      out_shape=[
          jax.ShapeDtypeStruct((npad, d), jnp.float32),
          jax.ShapeDtypeStruct((npad, L), jnp.float32),
      ],
  )(degp, x_pad)

  parts = _make_agg_kernel(npad, d, cpt)(xs, row3, col3)

  h_pad = pl.pallas_call(
      _head_kernel,
      grid=grid,
      in_specs=[
          pl.BlockSpec((bn, d), lambda i: (i, 0)),
          pl.BlockSpec((bn, d), lambda i: (i, 0)),
          pl.BlockSpec((bn, L), lambda i: (i, 0)),
          pl.BlockSpec((d, d), lambda i: (0, 0)),
          pl.BlockSpec((1, d), lambda i: (0, 0)),
      ],
      out_specs=pl.BlockSpec((bn, d), lambda i: (i, 0)),
      out_shape=jax.ShapeDtypeStruct((npad, d), jnp.float32),
  )(parts[:npad], parts[npad:], ds16, W, b.reshape(1, d))

  return h_pad[:n]


# speedup vs baseline: 1.5376x; 1.0795x over previous
import functools

import jax
import jax.numpy as jnp
from jax import lax
from jax.experimental import pallas as pl
from jax.experimental.pallas import tpu as pltpu
from jax.experimental.pallas import tpu_sc as plsc

NC = 2
NS = 16
L = 16
CH = 128


def _zero_rows(ref, nrows, width):
  zero = jnp.zeros((L,), jnp.float32)

  def body(i, carry):
    for j in range(width // L):
      ref[i, pl.ds(j * L, L)] = zero
    return carry

  lax.fori_loop(0, nrows, body, 0, unroll=4)


def _sc_mesh():
  return plsc.VectorSubcoreMesh(core_axis_name="c", subcore_axis_name="s")


def _make_deg_kernel(npad, cpt):
  @functools.partial(
      pl.kernel,
      out_type=jax.ShapeDtypeStruct((NC * NS, npad), jnp.float32),
      mesh=_sc_mesh(),
      compiler_params=pltpu.CompilerParams(needs_layout_passes=False),
      scratch_types=[
          pltpu.VMEM((cpt, CH), jnp.int32),
          pltpu.VMEM((npad,), jnp.float32),
      ],
  )
  def deg_kernel(col3, degp, colbuf, hist):
    c = lax.axis_index("c")
    s = lax.axis_index("s")
    wid = c * NS + s

    pltpu.sync_copy(col3.at[wid], colbuf)

    zero = jnp.zeros((L,), jnp.float32)

    def zbody(i, carry):
      hist[pl.ds(i * L, L)] = zero
      return carry

    lax.fori_loop(0, npad // L, zbody, 0, unroll=8)

    one = jnp.full((L,), 1.0, jnp.float32)

    def chunk(j, carry):
      for k in range(CH // L):
        idx = colbuf[j, pl.ds(k * L, L)]
        plsc.addupdate_scatter(hist, [idx], one)
      return carry

    lax.fori_loop(0, cpt, chunk, 0)
    pltpu.sync_copy(hist, degp.at[wid])

  return deg_kernel


def _make_agg_kernel(npad, d, cpt):
  rows_per_tile = npad // NS

  @functools.partial(
      pl.kernel,
      out_type=jax.ShapeDtypeStruct((NC * npad, d), jnp.float32),
      mesh=_sc_mesh(),
      scratch_types=[
          pltpu.VMEM((cpt, CH), jnp.int32),
          pltpu.VMEM((cpt, CH), jnp.int32),
          pltpu.VMEM((CH, d), jnp.float32),
          pltpu.VMEM_SHARED((npad, d), jnp.float32),
          pltpu.SemaphoreType.DMA,
      ],
  )
  def agg_kernel(xs_hbm, row3, col3, out, rowbuf, colbuf, rows_v, acc, sem):
    c = lax.axis_index("c")
    s = lax.axis_index("s")
    wid = c * NS + s

    pltpu.sync_copy(row3.at[wid], rowbuf)
    pltpu.sync_copy(col3.at[wid], colbuf)

    _zero_rows(rows_v, CH, d)
    zbase = s * rows_per_tile
    for r in range(rows_per_tile // CH):
      pltpu.sync_copy(rows_v, acc.at[pl.ds(zbase + r * CH, CH)])
    plsc.subcore_barrier()

    def chunk(j, carry):
      pltpu.async_copy(xs_hbm.at[rowbuf.at[j]], rows_v, sem).wait()
      pltpu.sync_copy(rows_v, acc.at[colbuf.at[j]], add=True)
      return carry

    lax.fori_loop(0, cpt, chunk, 0)
    plsc.subcore_barrier()

    pltpu.sync_copy(
        acc.at[pl.ds(s * rows_per_tile, rows_per_tile)],
        out.at[pl.ds(c * npad + s * rows_per_tile, rows_per_tile)])

  return agg_kernel


def _scale_kernel(dp_ref, x_ref, xs_ref, ds_ref):
  deg = jnp.sum(dp_ref[...], axis=0)[:, None]
  dis = jnp.where(deg > 0.0, lax.rsqrt(jnp.maximum(deg, 1e-30)), 0.0)
  xs_ref[...] = dis * x_ref[...]
  ds_ref[...] = jnp.broadcast_to(dis, ds_ref.shape)


def _head_kernel(p0_ref, p1_ref, ds_ref, w_ref, b_ref, o_ref):
  agg = ds_ref[:, :1] * (p0_ref[...] + p1_ref[...])
  out = jnp.dot(agg, w_ref[...], preferred_element_type=jnp.float32)
  o_ref[...] = jnp.maximum(out + b_ref[...], 0.0)


def kernel(x, edge_index, W, b):
  n, d = x.shape
  e = edge_index.shape[1]

  npad = -(-(n + 1) // (NS * CH)) * (NS * CH)
  cpt = -(-e // (NC * NS * CH))
  epad = NC * NS * cpt * CH

  row = edge_index[0].astype(jnp.int32)
  col = edge_index[1].astype(jnp.int32)
  pad = jnp.full((epad - e,), n, jnp.int32)
  row3 = jnp.concatenate([row, pad]).reshape(NC * NS, cpt, CH)
  col3 = jnp.concatenate([col, pad]).reshape(NC * NS, cpt, CH)
  x_pad = jnp.zeros((npad, d), x.dtype).at[:n].set(x)

  degp = _make_deg_kernel(npad, cpt)(col3)

  bn = 1024
  grid = (npad // bn,)
  xs, ds16 = pl.pallas_call(
      _scale_kernel,
      grid=grid,
      in_specs=[
          pl.BlockSpec((NC * NS, bn), lambda i: (0, i)),
          pl.BlockSpec((bn, d), lambda i: (i, 0)),
      ],
      out_specs=[
          pl.BlockSpec((bn, d), lambda i: (i, 0)),
          pl.BlockSpec((bn, L), lambda i: (i, 0)),
      ],
      out_shape=[
          jax.ShapeDtypeStruct((npad, d), jnp.float32),
          jax.ShapeDtypeStruct((npad, L), jnp.float32),
      ],
  )(degp, x_pad)

  parts = _make_agg_kernel(npad, d, cpt)(xs, row3, col3)

  h_pad = pl.pallas_call(
      _head_kernel,
      grid=grid,
      in_specs=[
          pl.BlockSpec((bn, d), lambda i: (i, 0)),
          pl.BlockSpec((bn, d), lambda i: (i, 0)),
          pl.BlockSpec((bn, L), lambda i: (i, 0)),
          pl.BlockSpec((d, d), lambda i: (0, 0)),
          pl.BlockSpec((1, d), lambda i: (0, 0)),
      ],
      out_specs=pl.BlockSpec((bn, d), lambda i: (i, 0)),
      out_shape=jax.ShapeDtypeStruct((npad, d), jnp.float32),
  )(parts[:npad], parts[npad:], ds16, W, b.reshape(1, d))

  return h_pad[:n]


# speedup vs baseline: 1.5533x; 1.0102x over previous
import functools

import jax
import jax.numpy as jnp
from jax import lax
from jax.experimental import pallas as pl
from jax.experimental.pallas import tpu as pltpu
from jax.experimental.pallas import tpu_sc as plsc

NC = 2
NS = 16
L = 16
CH = 128


def _zero_rows(ref, nrows, width):
  zero = jnp.zeros((L,), jnp.float32)

  def body(i, carry):
    for j in range(width // L):
      ref[i, pl.ds(j * L, L)] = zero
    return carry

  lax.fori_loop(0, nrows, body, 0, unroll=4)


def _sc_mesh():
  return plsc.VectorSubcoreMesh(core_axis_name="c", subcore_axis_name="s")


def _make_deg_kernel(npad, cpt):
  @functools.partial(
      pl.kernel,
      out_type=jax.ShapeDtypeStruct((NC * NS, npad), jnp.float32),
      mesh=_sc_mesh(),
      compiler_params=pltpu.CompilerParams(needs_layout_passes=False),
      scratch_types=[
          pltpu.VMEM((cpt, CH), jnp.int32),
          pltpu.VMEM((npad,), jnp.float32),
      ],
  )
  def deg_kernel(col3, degp, colbuf, hist):
    c = lax.axis_index("c")
    s = lax.axis_index("s")
    wid = c * NS + s

    pltpu.sync_copy(col3.at[wid], colbuf)

    zero = jnp.zeros((L,), jnp.float32)

    def zbody(i, carry):
      hist[pl.ds(i * L, L)] = zero
      return carry

    lax.fori_loop(0, npad // L, zbody, 0, unroll=8)

    one = jnp.full((L,), 1.0, jnp.float32)

    def chunk(j, carry):
      for k in range(CH // L):
        idx = colbuf[j, pl.ds(k * L, L)]
        plsc.addupdate_scatter(hist, [idx], one)
      return carry

    lax.fori_loop(0, cpt, chunk, 0)
    pltpu.sync_copy(hist, degp.at[wid])

  return deg_kernel


def _make_agg_kernel(npad, d, cpt):
  rows_per_tile = npad // NS

  @functools.partial(
      pl.kernel,
      out_type=jax.ShapeDtypeStruct((NC * npad, d), jnp.float32),
      mesh=_sc_mesh(),
      scratch_types=[
          pltpu.VMEM((cpt, CH), jnp.int32),
          pltpu.VMEM((cpt, CH), jnp.int32),
          pltpu.VMEM((CH, d), jnp.float32),
          pltpu.VMEM_SHARED((npad, d), jnp.float32),
          pltpu.SemaphoreType.DMA,
      ],
  )
  def agg_kernel(xs_hbm, row3, col3, out, rowbuf, colbuf, rows_v, acc, sem):
    c = lax.axis_index("c")
    s = lax.axis_index("s")
    wid = c * NS + s

    pltpu.sync_copy(row3.at[wid], rowbuf)
    pltpu.sync_copy(col3.at[wid], colbuf)

    _zero_rows(rows_v, CH, d)
    zbase = s * rows_per_tile
    for r in range(rows_per_tile // CH):
      pltpu.sync_copy(rows_v, acc.at[pl.ds(zbase + r * CH, CH)])
    plsc.subcore_barrier()

    def chunk(j, carry):
      pltpu.async_copy(xs_hbm.at[rowbuf.at[j]], rows_v, sem).wait()
      pltpu.sync_copy(rows_v, acc.at[colbuf.at[j]], add=True)
      return carry

    lax.fori_loop(0, cpt, chunk, 0)
    plsc.subcore_barrier()

    pltpu.sync_copy(
        acc.at[pl.ds(s * rows_per_tile, rows_per_tile)],
        out.at[pl.ds(c * npad + s * rows_per_tile, rows_per_tile)])

  return agg_kernel


def _scale_kernel(dp_ref, x_ref, xs_ref, ds_ref):
  deg = jnp.sum(dp_ref[...], axis=0)[:, None]
  dis = jnp.where(deg > 0.0, lax.rsqrt(jnp.maximum(deg, 1e-30)), 0.0)
  xs_ref[...] = dis * x_ref[...]
  ds_ref[...] = jnp.broadcast_to(dis, ds_ref.shape)


def _head_kernel(p0_ref, p1_ref, ds_ref, w_ref, b_ref, o_ref):
  agg = ds_ref[:, :1] * (p0_ref[...] + p1_ref[...])
  out = jnp.dot(agg, w_ref[...], preferred_element_type=jnp.float32)
  o_ref[...] = jnp.maximum(out + b_ref[...], 0.0)


def kernel(x, edge_index, W, b):
  n, d = x.shape
  e = edge_index.shape[1]

  npad = -(-(n + 1) // (NS * CH)) * (NS * CH)
  cpt = -(-e // (NC * NS * CH))
  epad = NC * NS * cpt * CH

  row = edge_index[0].astype(jnp.int32)
  col = edge_index[1].astype(jnp.int32)
  pad = jnp.full((epad - e,), n, jnp.int32)
  row3 = jnp.concatenate([row, pad]).reshape(NC * NS, cpt, CH)
  col3 = jnp.concatenate([col, pad]).reshape(NC * NS, cpt, CH)
  x_pad = jnp.zeros((npad, d), x.dtype).at[:n].set(x)

  degp = _make_deg_kernel(npad, cpt)(col3)

  bn = 2048
  grid = (npad // bn,)
  xs, ds16 = pl.pallas_call(
      _scale_kernel,
      grid=grid,
      in_specs=[
          pl.BlockSpec((NC * NS, bn), lambda i: (0, i)),
          pl.BlockSpec((bn, d), lambda i: (i, 0)),
      ],
      out_specs=[
          pl.BlockSpec((bn, d), lambda i: (i, 0)),
          pl.BlockSpec((bn, L), lambda i: (i, 0)),
      ],
      out_shape=[
          jax.ShapeDtypeStruct((npad, d), jnp.float32),
          jax.ShapeDtypeStruct((npad, L), jnp.float32),
      ],
  )(degp, x_pad)

  parts = _make_agg_kernel(npad, d, cpt)(xs, row3, col3)

  h_pad = pl.pallas_call(
      _head_kernel,
      grid=grid,
      in_specs=[
          pl.BlockSpec((bn, d), lambda i: (i, 0)),
          pl.BlockSpec((bn, d), lambda i: (i, 0)),
          pl.BlockSpec((bn, L), lambda i: (i, 0)),
          pl.BlockSpec((d, d), lambda i: (0, 0)),
          pl.BlockSpec((1, d), lambda i: (0, 0)),
      ],
      out_specs=pl.BlockSpec((bn, d), lambda i: (i, 0)),
      out_shape=jax.ShapeDtypeStruct((npad, d), jnp.float32),
  )(parts[:npad], parts[npad:], ds16, W, b.reshape(1, d))

  return h_pad[:n]


# speedup vs baseline: 1.5581x; 1.0031x over previous
import functools

import jax
import jax.numpy as jnp
from jax import lax
from jax.experimental import pallas as pl
from jax.experimental.pallas import tpu as pltpu
from jax.experimental.pallas import tpu_sc as plsc

NC = 2
NS = 16
L = 16
CH = 128


def _zero_rows(ref, nrows, width):
  zero = jnp.zeros((L,), jnp.float32)

  def body(i, carry):
    for j in range(width // L):
      ref[i, pl.ds(j * L, L)] = zero
    return carry

  lax.fori_loop(0, nrows, body, 0, unroll=4)


def _sc_mesh():
  return plsc.VectorSubcoreMesh(core_axis_name="c", subcore_axis_name="s")


def _make_deg_kernel(npad, cpt):
  @functools.partial(
      pl.kernel,
      out_type=jax.ShapeDtypeStruct((NC * NS, npad), jnp.float32),
      mesh=_sc_mesh(),
      compiler_params=pltpu.CompilerParams(needs_layout_passes=False),
      scratch_types=[
          pltpu.VMEM((cpt, CH), jnp.int32),
          pltpu.VMEM((npad,), jnp.float32),
      ],
  )
  def deg_kernel(col3, degp, colbuf, hist):
    c = lax.axis_index("c")
    s = lax.axis_index("s")
    wid = c * NS + s

    pltpu.sync_copy(col3.at[wid], colbuf)

    zero = jnp.zeros((L,), jnp.float32)

    def zbody(i, carry):
      hist[pl.ds(i * L, L)] = zero
      return carry

    lax.fori_loop(0, npad // L, zbody, 0, unroll=8)

    one = jnp.full((L,), 1.0, jnp.float32)

    def chunk(j, carry):
      for k in range(CH // L):
        idx = colbuf[j, pl.ds(k * L, L)]
        plsc.addupdate_scatter(hist, [idx], one)
      return carry

    lax.fori_loop(0, cpt, chunk, 0)
    pltpu.sync_copy(hist, degp.at[wid])

  return deg_kernel


def _make_agg_kernel(npad, d, cpt):
  rows_per_tile = npad // NS

  @functools.partial(
      pl.kernel,
      out_type=jax.ShapeDtypeStruct((NC * npad, d), jnp.float32),
      mesh=_sc_mesh(),
      scratch_types=[
          pltpu.VMEM((cpt, CH), jnp.int32),
          pltpu.VMEM((cpt, CH), jnp.int32),
          pltpu.VMEM((CH, d), jnp.float32),
          pltpu.VMEM_SHARED((npad, d), jnp.float32),
          pltpu.SemaphoreType.DMA,
      ],
  )
  def agg_kernel(xs_hbm, row3, col3, out, rowbuf, colbuf, rows_v, acc, sem):
    c = lax.axis_index("c")
    s = lax.axis_index("s")
    wid = c * NS + s

    pltpu.sync_copy(row3.at[wid], rowbuf)
    pltpu.sync_copy(col3.at[wid], colbuf)

    _zero_rows(rows_v, CH, d)
    zbase = s * rows_per_tile
    for r in range(rows_per_tile // CH):
      pltpu.sync_copy(rows_v, acc.at[pl.ds(zbase + r * CH, CH)])
    plsc.subcore_barrier()

    def chunk(j, carry):
      pltpu.async_copy(xs_hbm.at[rowbuf.at[j]], rows_v, sem).wait()
      pltpu.sync_copy(rows_v, acc.at[colbuf.at[j]], add=True)
      return carry

    lax.fori_loop(0, cpt, chunk, 0)
    plsc.subcore_barrier()

    pltpu.sync_copy(
        acc.at[pl.ds(s * rows_per_tile, rows_per_tile)],
        out.at[pl.ds(c * npad + s * rows_per_tile, rows_per_tile)])

  return agg_kernel


def _scale_kernel(dp_ref, x_ref, xs_ref, ds_ref):
  deg = jnp.sum(dp_ref[...], axis=0)[:, None]
  dis = jnp.where(deg > 0.0, lax.rsqrt(jnp.maximum(deg, 1e-30)), 0.0)
  xs_ref[...] = dis * x_ref[...]
  ds_ref[...] = jnp.broadcast_to(dis, ds_ref.shape)


def _head_kernel(p0_ref, p1_ref, ds_ref, w_ref, b_ref, o_ref):
  agg = ds_ref[:, :1] * (p0_ref[...] + p1_ref[...])
  out = jnp.dot(agg, w_ref[...], preferred_element_type=jnp.float32)
  o_ref[...] = jnp.maximum(out + b_ref[...], 0.0)


def kernel(x, edge_index, W, b):
  n, d = x.shape
  e = edge_index.shape[1]

  npad = -(-(n + 1) // (NS * CH)) * (NS * CH)
  cpt = -(-e // (NC * NS * CH))
  epad = NC * NS * cpt * CH

  row = edge_index[0].astype(jnp.int32)
  col = edge_index[1].astype(jnp.int32)
  pad = jnp.full((epad - e,), n, jnp.int32)
  row3 = jnp.concatenate([row, pad]).reshape(NC * NS, cpt, CH)
  col3 = jnp.concatenate([col, pad]).reshape(NC * NS, cpt, CH)
  x_pad = jnp.zeros((npad, d), x.dtype).at[:n].set(x)

  degp = _make_deg_kernel(npad, cpt)(col3)

  bn = npad
  grid = (npad // bn,)
  xs, ds16 = pl.pallas_call(
      _scale_kernel,
      grid=grid,
      in_specs=[
          pl.BlockSpec((NC * NS, bn), lambda i: (0, i)),
          pl.BlockSpec((bn, d), lambda i: (i, 0)),
      ],
      out_specs=[
          pl.BlockSpec((bn, d), lambda i: (i, 0)),
          pl.BlockSpec((bn, L), lambda i: (i, 0)),
      ],
      out_shape=[
          jax.ShapeDtypeStruct((npad, d), jnp.float32),
          jax.ShapeDtypeStruct((npad, L), jnp.float32),
      ],
  )(degp, x_pad)

  parts = _make_agg_kernel(npad, d, cpt)(xs, row3, col3)

  h_pad = pl.pallas_call(
      _head_kernel,
      grid=grid,
      in_specs=[
          pl.BlockSpec((bn, d), lambda i: (i, 0)),
          pl.BlockSpec((bn, d), lambda i: (i, 0)),
          pl.BlockSpec((bn, L), lambda i: (i, 0)),
          pl.BlockSpec((d, d), lambda i: (0, 0)),
          pl.BlockSpec((1, d), lambda i: (0, 0)),
      ],
      out_specs=pl.BlockSpec((bn, d), lambda i: (i, 0)),
      out_shape=jax.ShapeDtypeStruct((npad, d), jnp.float32),
  )(parts[:npad], parts[npad:], ds16, W, b.reshape(1, d))

  return h_pad[:n]
